# Initial kernel scaffold; baseline (speedup 1.0000x reference)
import jax
import jax.numpy as jnp
from jax.experimental import pallas as pl


def kernel(X_input, edge_index, W1, b1, W2, b2, Wa, ba, Wd1, bd1, Wd2, bd2, Wd3, bd3, Wm, bm, Ws, bs, Wp, bp, mu):
    raise NotImplementedError("write your pallas kernel here")


# speedup vs baseline: 1.1668x; 1.1668x over previous
import functools
import jax
import jax.numpy as jnp
from jax.experimental import pallas as pl
from jax.experimental.pallas import tpu as pltpu

N = 10000
E = 160000
IN_DIM = 256
HID = 128
LAT = 15
ADJ_DIM = 32
K = 3
NCLUST = 10

ROWS_B = 400


def _heads_body(z_ref, wa_ref, ba_ref, wd1_ref, bd1_ref, wd2_ref, bd2_ref,
                wd3_ref, bd3_ref, wm_ref, bm_ref, ws_ref, bs_ref, wp_ref,
                bp_ref, mu_ref, dech_ref, q_ref, mean_ref, disp_ref, pi_ref):
    z = z_ref[...]
    f32 = jnp.float32
    dot = functools.partial(jnp.dot, preferred_element_type=f32)
    dech_ref[...] = dot(z, wa_ref[...]) + ba_ref[...]
    h = jax.nn.relu(dot(z, wd1_ref[...]) + bd1_ref[...])
    h = jax.nn.relu(dot(h, wd2_ref[...]) + bd2_ref[...])
    h = jax.nn.relu(dot(h, wd3_ref[...]) + bd3_ref[...])
    mean_ref[...] = jnp.clip(jnp.exp(dot(h, wm_ref[...]) + bm_ref[...]),
                             1e-5, 1e6)
    disp_ref[...] = jnp.clip(jax.nn.softplus(dot(h, ws_ref[...]) + bs_ref[...]),
                             1e-4, 1e4)
    pi_ref[...] = jax.nn.sigmoid(dot(h, wp_ref[...]) + bp_ref[...])
    mu = mu_ref[...]
    cols = []
    for j in range(NCLUST):
        d = z - mu[j][None, :]
        cols.append(jnp.sum(d * d, axis=1, keepdims=True))
    dist = jnp.concatenate(cols, axis=1)
    q = 1.0 / (1.0 + dist)
    q_ref[...] = q / jnp.sum(q, axis=1, keepdims=True)


def _adj_body(dech_blk_ref, dech_all_ref, out_ref):
    out_ref[...] = jax.nn.sigmoid(
        jnp.dot(dech_blk_ref[...], dech_all_ref[...].T,
                preferred_element_type=jnp.float32))


def _decoder_heads(z, Wa, ba, Wd1, bd1, Wd2, bd2, Wd3, bd3, Wm, bm, Ws, bs,
                   Wp, bp, mu):
    grid = (N // ROWS_B,)
    row_spec = lambda d: pl.BlockSpec((ROWS_B, d), lambda i: (i, 0))
    full = lambda a: pl.BlockSpec(a.shape, lambda i: tuple(0 for _ in a.shape))
    out_shapes = (
        jax.ShapeDtypeStruct((N, ADJ_DIM), jnp.float32),
        jax.ShapeDtypeStruct((N, NCLUST), jnp.float32),
        jax.ShapeDtypeStruct((N, IN_DIM), jnp.float32),
        jax.ShapeDtypeStruct((N, IN_DIM), jnp.float32),
        jax.ShapeDtypeStruct((N, IN_DIM), jnp.float32),
    )
    consts = (Wa, ba, Wd1, bd1, Wd2, bd2, Wd3, bd3, Wm, bm, Ws, bs, Wp, bp, mu)
    return pl.pallas_call(
        _heads_body,
        grid=grid,
        in_specs=[row_spec(LAT)] + [full(c) for c in consts],
        out_specs=(row_spec(ADJ_DIM), row_spec(NCLUST), row_spec(IN_DIM),
                   row_spec(IN_DIM), row_spec(IN_DIM)),
        out_shape=out_shapes,
    )(z, *consts)


def _adj_out(dec_h):
    grid = (N // ROWS_B,)
    return pl.pallas_call(
        _adj_body,
        grid=grid,
        in_specs=[pl.BlockSpec((ROWS_B, ADJ_DIM), lambda i: (i, 0)),
                  pl.BlockSpec((N, ADJ_DIM), lambda i: (0, 0))],
        out_specs=pl.BlockSpec((ROWS_B, N), lambda i: (i, 0)),
        out_shape=jax.ShapeDtypeStruct((N, N), jnp.float32),
    )(dec_h, dec_h)


def _propagate(v, src, dst, norm):
    h = v * norm[:, None]
    h = jax.ops.segment_sum(h[src], dst, num_segments=N)
    return h * norm[:, None]


def _tag_layer(x, src, dst, norm, W, b, out_dim):
    in_dim = x.shape[1]
    Wh = W.reshape(K + 1, in_dim, out_dim)
    Wcat = jnp.concatenate([Wh[i] for i in range(K + 1)], axis=1)
    T = x @ Wcat
    acc = T[:, K * out_dim:(K + 1) * out_dim]
    for i in range(K - 1, -1, -1):
        acc = _propagate(acc, src, dst, norm)
        acc = acc + T[:, i * out_dim:(i + 1) * out_dim]
    return acc + b


def kernel(X_input, edge_index, W1, b1, W2, b2, Wa, ba, Wd1, bd1, Wd2, bd2,
           Wd3, bd3, Wm, bm, Ws, bs, Wp, bp, mu):
    src, dst = edge_index[0], edge_index[1]
    deg = jax.ops.segment_sum(jnp.ones(E, jnp.float32), dst, num_segments=N)
    norm = jnp.where(deg > 0.0, deg, 1.0) ** -0.5
    enc_h = _tag_layer(X_input, src, dst, norm, W1, b1, HID)
    z = _tag_layer(enc_h, src, dst, norm, W2, b2, LAT)
    dec_h, q, _mean, _disp, _pi = _decoder_heads(
        z, Wa, ba, Wd1, bd1, Wd2, bd2, Wd3, bd3, Wm, bm, Ws, bs, Wp, bp, mu)
    A_out = _adj_out(dec_h)
    return (A_out, z, q, _mean, _disp, _pi)


# speedup vs baseline: 4.1011x; 3.5149x over previous
import functools
import jax
import jax.numpy as jnp
from jax import lax
from jax.experimental import pallas as pl
from jax.experimental.pallas import tpu as pltpu
from jax.experimental.pallas import tpu_sc as plsc

N = 10000
E = 160000
IN_DIM = 256
HID = 128
LAT = 15
ADJ_DIM = 32
K = 3
NCLUST = 10

NC = 2
NS = 16
NW = NC * NS
EB = 128
CHUNKS_W = 40
EP = NW * CHUNKS_W * EB
NP = 10240
ROWS_PER_SUB = NP // NS
ZROWS = 128

ROWS_B = 400


def _sc_segsum(v_pad, srcp, dstp, zrows):
    D = v_pad.shape[1]
    mesh = plsc.VectorSubcoreMesh(core_axis_name="c", subcore_axis_name="s",
                                  num_cores=NC, num_subcores=NS)

    @functools.partial(
        pl.kernel,
        out_type=jax.ShapeDtypeStruct((NC, NP, D), jnp.float32),
        mesh=mesh,
        scratch_types=[
            pltpu.VMEM((CHUNKS_W, EB), jnp.int32),
            pltpu.VMEM((CHUNKS_W, EB), jnp.int32),
            pltpu.VMEM((EB, D), jnp.float32),
            pltpu.VMEM_SHARED((NP, D), jnp.float32),
            pltpu.SemaphoreType.DMA,
        ],
        compiler_params=pltpu.CompilerParams(use_tc_tiling_on_sc=False),
    )
    def seg_kernel(v_hbm, src_hbm, dst_hbm, z_hbm, out_hbm,
                   src_v, dst_v, rows_v, acc_sh, sem):
        c = lax.axis_index("c")
        s = lax.axis_index("s")
        w = s * NC + c
        row0 = s * ROWS_PER_SUB
        for j in range(ROWS_PER_SUB // ZROWS):
            pltpu.sync_copy(z_hbm, acc_sh.at[pl.ds(row0 + j * ZROWS, ZROWS)])
        pltpu.sync_copy(src_hbm.at[pl.ds(w * CHUNKS_W, CHUNKS_W)], src_v)
        pltpu.sync_copy(dst_hbm.at[pl.ds(w * CHUNKS_W, CHUNKS_W)], dst_v)
        plsc.subcore_barrier()

        @pl.loop(0, CHUNKS_W)
        def _(k):
            pltpu.async_copy(v_hbm.at[src_v.at[k]], rows_v, sem).wait()
            pltpu.sync_copy(rows_v, acc_sh.at[dst_v.at[k]], add=True)

        plsc.subcore_barrier()
        pltpu.sync_copy(acc_sh.at[pl.ds(row0, ROWS_PER_SUB)],
                        out_hbm.at[c].at[pl.ds(row0, ROWS_PER_SUB)])

    return seg_kernel(v_pad, srcp, dstp, zrows)


def _propagate(v, srcp, dstp, zrows, norm):
    vpre = v * norm[:, None]
    v_pad = jnp.concatenate(
        [vpre, jnp.zeros((NP - N, v.shape[1]), jnp.float32)], axis=0)
    parts = _sc_segsum(v_pad, srcp, dstp, zrows)
    return (parts[0, :N] + parts[1, :N]) * norm[:, None]


def _mm_body(x_ref, w_ref, o_ref):
    o_ref[...] = jnp.dot(x_ref[...], w_ref[...],
                         preferred_element_type=jnp.float32)


def _mm(x, w):
    n, din = x.shape
    dout = w.shape[1]
    return pl.pallas_call(
        _mm_body,
        grid=(n // ROWS_B,),
        in_specs=[pl.BlockSpec((ROWS_B, din), lambda i: (i, 0)),
                  pl.BlockSpec((din, dout), lambda i: (0, 0))],
        out_specs=pl.BlockSpec((ROWS_B, dout), lambda i: (i, 0)),
        out_shape=jax.ShapeDtypeStruct((n, dout), jnp.float32),
    )(x, w)


def _heads_body(z_ref, wa_ref, ba_ref, wd1_ref, bd1_ref, wd2_ref, bd2_ref,
                wd3_ref, bd3_ref, wm_ref, bm_ref, ws_ref, bs_ref, wp_ref,
                bp_ref, mu_ref, dech_ref, q_ref, mean_ref, disp_ref, pi_ref):
    z = z_ref[...]
    f32 = jnp.float32
    dot = functools.partial(jnp.dot, preferred_element_type=f32)
    dech_ref[...] = dot(z, wa_ref[...]) + ba_ref[...]
    h = jax.nn.relu(dot(z, wd1_ref[...]) + bd1_ref[...])
    h = jax.nn.relu(dot(h, wd2_ref[...]) + bd2_ref[...])
    h = jax.nn.relu(dot(h, wd3_ref[...]) + bd3_ref[...])
    mean_ref[...] = jnp.clip(jnp.exp(dot(h, wm_ref[...]) + bm_ref[...]),
                             1e-5, 1e6)
    disp_ref[...] = jnp.clip(jax.nn.softplus(dot(h, ws_ref[...]) + bs_ref[...]),
                             1e-4, 1e4)
    pi_ref[...] = jax.nn.sigmoid(dot(h, wp_ref[...]) + bp_ref[...])
    mu = mu_ref[...]
    cols = []
    for j in range(NCLUST):
        d = z - mu[j][None, :]
        cols.append(jnp.sum(d * d, axis=1, keepdims=True))
    dist = jnp.concatenate(cols, axis=1)
    q = 1.0 / (1.0 + dist)
    q_ref[...] = q / jnp.sum(q, axis=1, keepdims=True)


def _adj_body(dech_blk_ref, dech_all_ref, out_ref):
    out_ref[...] = jax.nn.sigmoid(
        jnp.dot(dech_blk_ref[...], dech_all_ref[...].T,
                preferred_element_type=jnp.float32))


def _decoder_heads(z, Wa, ba, Wd1, bd1, Wd2, bd2, Wd3, bd3, Wm, bm, Ws, bs,
                   Wp, bp, mu):
    grid = (N // ROWS_B,)
    row_spec = lambda d: pl.BlockSpec((ROWS_B, d), lambda i: (i, 0))
    full = lambda a: pl.BlockSpec(a.shape, lambda i: tuple(0 for _ in a.shape))
    out_shapes = (
        jax.ShapeDtypeStruct((N, ADJ_DIM), jnp.float32),
        jax.ShapeDtypeStruct((N, NCLUST), jnp.float32),
        jax.ShapeDtypeStruct((N, IN_DIM), jnp.float32),
        jax.ShapeDtypeStruct((N, IN_DIM), jnp.float32),
        jax.ShapeDtypeStruct((N, IN_DIM), jnp.float32),
    )
    consts = (Wa, ba, Wd1, bd1, Wd2, bd2, Wd3, bd3, Wm, bm, Ws, bs, Wp, bp, mu)
    return pl.pallas_call(
        _heads_body,
        grid=grid,
        in_specs=[row_spec(LAT)] + [full(c) for c in consts],
        out_specs=(row_spec(ADJ_DIM), row_spec(NCLUST), row_spec(IN_DIM),
                   row_spec(IN_DIM), row_spec(IN_DIM)),
        out_shape=out_shapes,
    )(z, *consts)


def _adj_out(dec_h):
    grid = (N // ROWS_B,)
    return pl.pallas_call(
        _adj_body,
        grid=grid,
        in_specs=[pl.BlockSpec((ROWS_B, ADJ_DIM), lambda i: (i, 0)),
                  pl.BlockSpec((N, ADJ_DIM), lambda i: (0, 0))],
        out_specs=pl.BlockSpec((ROWS_B, N), lambda i: (i, 0)),
        out_shape=jax.ShapeDtypeStruct((N, N), jnp.float32),
    )(dec_h, dec_h)


def _tag_layer(x, srcp, dstp, zrows, norm, W, b, out_dim, pad_dim):
    in_dim = x.shape[1]
    Wh = W.reshape(K + 1, in_dim, out_dim)
    if pad_dim != out_dim:
        pad = jnp.zeros((in_dim, pad_dim - out_dim), jnp.float32)
        blocks = [jnp.concatenate([Wh[i], pad], axis=1) for i in range(K + 1)]
    else:
        blocks = [Wh[i] for i in range(K + 1)]
    Wcat = jnp.concatenate(blocks, axis=1)
    T = _mm(x, Wcat)
    acc = T[:, K * pad_dim:(K + 1) * pad_dim]
    for i in range(K - 1, -1, -1):
        acc = _propagate(acc, srcp, dstp, zrows, norm)
        acc = acc + T[:, i * pad_dim:(i + 1) * pad_dim]
    return (acc[:, :out_dim] + b) if pad_dim != out_dim else (acc + b)


def kernel(X_input, edge_index, W1, b1, W2, b2, Wa, ba, Wd1, bd1, Wd2, bd2,
           Wd3, bd3, Wm, bm, Ws, bs, Wp, bp, mu):
    src, dst = edge_index[0], edge_index[1]
    padlen = EP - E
    srcp = jnp.concatenate(
        [src.astype(jnp.int32), jnp.full((padlen,), N, jnp.int32)]
    ).reshape(NW * CHUNKS_W, EB)
    dstp = jnp.concatenate(
        [dst.astype(jnp.int32), jnp.full((padlen,), N, jnp.int32)]
    ).reshape(NW * CHUNKS_W, EB)

    zrows128 = jnp.zeros((ZROWS, HID), jnp.float32)
    zrows16 = jnp.zeros((ZROWS, 16), jnp.float32)

    vones = jnp.concatenate(
        [jnp.ones((N, 16), jnp.float32), jnp.zeros((NP - N, 16), jnp.float32)],
        axis=0)
    dparts = _sc_segsum(vones, srcp, dstp, zrows16)
    deg = dparts[0, :N, 0] + dparts[1, :N, 0]
    norm = jnp.where(deg > 0.0, deg, 1.0) ** -0.5

    enc_h = _tag_layer(X_input, srcp, dstp, zrows128, norm, W1, b1, HID, HID)
    z = _tag_layer(enc_h, srcp, dstp, zrows16, norm, W2, b2, LAT, 16)

    dec_h, q, _mean, _disp, _pi = _decoder_heads(
        z, Wa, ba, Wd1, bd1, Wd2, bd2, Wd3, bd3, Wm, bm, Ws, bs, Wp, bp, mu)
    A_out = _adj_out(dec_h)
    return (A_out, z, q, _mean, _disp, _pi)


# speedup vs baseline: 4.2832x; 1.0444x over previous
import functools
import jax
import jax.numpy as jnp
from jax import lax
from jax.experimental import pallas as pl
from jax.experimental.pallas import tpu as pltpu
from jax.experimental.pallas import tpu_sc as plsc

N = 10000
E = 160000
IN_DIM = 256
HID = 128
LAT = 15
ADJ_DIM = 32
K = 3
NCLUST = 10

NC = 2
NS = 16
NW = NC * NS
EB = 128
CHUNKS_W = 40
EP = NW * CHUNKS_W * EB
NP = 10240
ROWS_PER_SUB = NP // NS
ZROWS = 128

ROWS_B = 400


@functools.lru_cache(maxsize=None)
def _sc_segsum_kernel(D):
    NBUF = 2 if D >= 64 else 4
    mesh = plsc.VectorSubcoreMesh(core_axis_name="c", subcore_axis_name="s",
                                  num_cores=NC, num_subcores=NS)

    @functools.partial(
        pl.kernel,
        out_type=jax.ShapeDtypeStruct((NC, NP, D), jnp.float32),
        mesh=mesh,
        scratch_types=[
            pltpu.VMEM((CHUNKS_W, EB), jnp.int32),
            pltpu.VMEM((CHUNKS_W, EB), jnp.int32),
            pltpu.VMEM((NBUF, EB, D), jnp.float32),
            pltpu.VMEM_SHARED((NP, D), jnp.float32),
            pltpu.SemaphoreType.DMA((NBUF,)),
            pltpu.SemaphoreType.DMA((NBUF,)),
        ],
        compiler_params=pltpu.CompilerParams(use_tc_tiling_on_sc=False),
    )
    def seg_kernel(v_hbm, src_hbm, dst_hbm, z_hbm, out_hbm,
                   src_v, dst_v, rows, acc_sh, gsem, ssem):
        c = lax.axis_index("c")
        s = lax.axis_index("s")
        w = s * NC + c
        row0 = s * ROWS_PER_SUB
        nz = ROWS_PER_SUB // ZROWS
        for j in range(nz):
            pltpu.async_copy(z_hbm, acc_sh.at[pl.ds(row0 + j * ZROWS, ZROWS)],
                             gsem.at[j % NBUF])
        pltpu.async_copy(src_hbm.at[pl.ds(w * CHUNKS_W, CHUNKS_W)], src_v,
                         ssem.at[0])
        pltpu.async_copy(dst_hbm.at[pl.ds(w * CHUNKS_W, CHUNKS_W)], dst_v,
                         ssem.at[1])
        for j in range(nz):
            pltpu.make_async_copy(
                z_hbm, acc_sh.at[pl.ds(row0 + j * ZROWS, ZROWS)],
                gsem.at[j % NBUF]).wait()
        pltpu.make_async_copy(src_hbm.at[pl.ds(w * CHUNKS_W, CHUNKS_W)],
                              src_v, ssem.at[0]).wait()
        pltpu.make_async_copy(dst_hbm.at[pl.ds(w * CHUNKS_W, CHUNKS_W)],
                              dst_v, ssem.at[1]).wait()
        plsc.subcore_barrier()

        dummy = v_hbm.at[pl.ds(0, EB)]
        for b in range(NBUF):
            pltpu.async_copy(v_hbm.at[src_v.at[b]], rows.at[b], gsem.at[b])

        nblk = CHUNKS_W // NBUF

        @pl.loop(0, nblk)
        def _(k0):
            kb = k0 * NBUF
            for b in range(NBUF):
                pltpu.make_async_copy(dummy, rows.at[b], gsem.at[b]).wait()
                pltpu.async_copy(rows.at[b], acc_sh.at[dst_v.at[kb + b]],
                                 ssem.at[b], add=True)
            for b in range(NBUF):
                @pl.when(k0 < nblk - 1)
                def _():
                    pltpu.make_async_copy(dummy, rows.at[b], ssem.at[b]).wait()
                    pltpu.async_copy(v_hbm.at[src_v.at[kb + NBUF + b]],
                                     rows.at[b], gsem.at[b])

        for b in range(NBUF):
            pltpu.make_async_copy(dummy, rows.at[b], ssem.at[b]).wait()
        plsc.subcore_barrier()
        pltpu.sync_copy(acc_sh.at[pl.ds(row0, ROWS_PER_SUB)],
                        out_hbm.at[c].at[pl.ds(row0, ROWS_PER_SUB)])

    return seg_kernel


def _sc_segsum(v_pad, srcp, dstp, zrows):
    return _sc_segsum_kernel(v_pad.shape[1])(v_pad, srcp, dstp, zrows)


@functools.lru_cache(maxsize=None)
def _sc_degree_kernel():
    mesh = plsc.VectorSubcoreMesh(core_axis_name="c", subcore_axis_name="s",
                                  num_cores=NC, num_subcores=NS)

    @functools.partial(
        pl.kernel,
        out_type=jax.ShapeDtypeStruct((NC, NP, 16), jnp.float32),
        mesh=mesh,
        scratch_types=[
            pltpu.VMEM((CHUNKS_W, EB), jnp.int32),
            pltpu.VMEM((EB, 16), jnp.float32),
            pltpu.VMEM_SHARED((NP, 16), jnp.float32),
            pltpu.SemaphoreType.DMA,
        ],
        compiler_params=pltpu.CompilerParams(use_tc_tiling_on_sc=False),
    )
    def deg_kernel(dst_hbm, z_hbm, out_hbm, dst_v, ones_v, acc_sh, sem):
        c = lax.axis_index("c")
        s = lax.axis_index("s")
        w = s * NC + c
        row0 = s * ROWS_PER_SUB

        @pl.loop(0, EB)
        def _(r):
            ones_v.at[pl.ds(r, 1), :][...] = jnp.ones((1, 16), jnp.float32)

        for j in range(ROWS_PER_SUB // ZROWS):
            pltpu.sync_copy(z_hbm, acc_sh.at[pl.ds(row0 + j * ZROWS, ZROWS)])
        pltpu.sync_copy(dst_hbm.at[pl.ds(w * CHUNKS_W, CHUNKS_W)], dst_v)
        plsc.subcore_barrier()

        @pl.loop(0, CHUNKS_W)
        def _(k):
            pltpu.async_copy(ones_v, acc_sh.at[dst_v.at[k]], sem, add=True)

        @pl.loop(0, CHUNKS_W)
        def _(k):
            pltpu.make_async_copy(ones_v, acc_sh.at[dst_v.at[k]], sem).wait()

        plsc.subcore_barrier()
        pltpu.sync_copy(acc_sh.at[pl.ds(row0, ROWS_PER_SUB)],
                        out_hbm.at[c].at[pl.ds(row0, ROWS_PER_SUB)])

    return deg_kernel


def _sc_degree(dstp, zrows16):
    return _sc_degree_kernel()(dstp, zrows16)


def _propagate(v, srcp, dstp, zrows, norm):
    vpre = v * norm[:, None]
    v_pad = jnp.concatenate(
        [vpre, jnp.zeros((NP - N, v.shape[1]), jnp.float32)], axis=0)
    parts = _sc_segsum(v_pad, srcp, dstp, zrows)
    return (parts[0, :N] + parts[1, :N]) * norm[:, None]


def _mm_body(x_ref, w_ref, o_ref):
    o_ref[...] = jnp.dot(x_ref[...], w_ref[...],
                         preferred_element_type=jnp.float32)


def _mm(x, w):
    n, din = x.shape
    dout = w.shape[1]
    return pl.pallas_call(
        _mm_body,
        grid=(n // ROWS_B,),
        in_specs=[pl.BlockSpec((ROWS_B, din), lambda i: (i, 0)),
                  pl.BlockSpec((din, dout), lambda i: (0, 0))],
        out_specs=pl.BlockSpec((ROWS_B, dout), lambda i: (i, 0)),
        out_shape=jax.ShapeDtypeStruct((n, dout), jnp.float32),
    )(x, w)


def _heads_body(z_ref, wa_ref, ba_ref, wd1_ref, bd1_ref, wd2_ref, bd2_ref,
                wd3_ref, bd3_ref, wm_ref, bm_ref, ws_ref, bs_ref, wp_ref,
                bp_ref, mu_ref, dech_ref, q_ref, mean_ref, disp_ref, pi_ref):
    z = z_ref[...]
    f32 = jnp.float32
    dot = functools.partial(jnp.dot, preferred_element_type=f32)
    dech_ref[...] = dot(z, wa_ref[...]) + ba_ref[...]
    h = jax.nn.relu(dot(z, wd1_ref[...]) + bd1_ref[...])
    h = jax.nn.relu(dot(h, wd2_ref[...]) + bd2_ref[...])
    h = jax.nn.relu(dot(h, wd3_ref[...]) + bd3_ref[...])
    mean_ref[...] = jnp.clip(jnp.exp(dot(h, wm_ref[...]) + bm_ref[...]),
                             1e-5, 1e6)
    disp_ref[...] = jnp.clip(jax.nn.softplus(dot(h, ws_ref[...]) + bs_ref[...]),
                             1e-4, 1e4)
    pi_ref[...] = jax.nn.sigmoid(dot(h, wp_ref[...]) + bp_ref[...])
    mu = mu_ref[...]
    cols = []
    for j in range(NCLUST):
        d = z - mu[j][None, :]
        cols.append(jnp.sum(d * d, axis=1, keepdims=True))
    dist = jnp.concatenate(cols, axis=1)
    q = 1.0 / (1.0 + dist)
    q_ref[...] = q / jnp.sum(q, axis=1, keepdims=True)


def _adj_body(dech_blk_ref, dech_all_ref, out_ref):
    out_ref[...] = jax.nn.sigmoid(
        jnp.dot(dech_blk_ref[...], dech_all_ref[...].T,
                preferred_element_type=jnp.float32))


def _decoder_heads(z, Wa, ba, Wd1, bd1, Wd2, bd2, Wd3, bd3, Wm, bm, Ws, bs,
                   Wp, bp, mu):
    grid = (N // ROWS_B,)
    row_spec = lambda d: pl.BlockSpec((ROWS_B, d), lambda i: (i, 0))
    full = lambda a: pl.BlockSpec(a.shape, lambda i: tuple(0 for _ in a.shape))
    out_shapes = (
        jax.ShapeDtypeStruct((N, ADJ_DIM), jnp.float32),
        jax.ShapeDtypeStruct((N, NCLUST), jnp.float32),
        jax.ShapeDtypeStruct((N, IN_DIM), jnp.float32),
        jax.ShapeDtypeStruct((N, IN_DIM), jnp.float32),
        jax.ShapeDtypeStruct((N, IN_DIM), jnp.float32),
    )
    consts = (Wa, ba, Wd1, bd1, Wd2, bd2, Wd3, bd3, Wm, bm, Ws, bs, Wp, bp, mu)
    return pl.pallas_call(
        _heads_body,
        grid=grid,
        in_specs=[row_spec(LAT)] + [full(c) for c in consts],
        out_specs=(row_spec(ADJ_DIM), row_spec(NCLUST), row_spec(IN_DIM),
                   row_spec(IN_DIM), row_spec(IN_DIM)),
        out_shape=out_shapes,
    )(z, *consts)


def _adj_out(dec_h):
    grid = (N // ROWS_B,)
    return pl.pallas_call(
        _adj_body,
        grid=grid,
        in_specs=[pl.BlockSpec((ROWS_B, ADJ_DIM), lambda i: (i, 0)),
                  pl.BlockSpec((N, ADJ_DIM), lambda i: (0, 0))],
        out_specs=pl.BlockSpec((ROWS_B, N), lambda i: (i, 0)),
        out_shape=jax.ShapeDtypeStruct((N, N), jnp.float32),
    )(dec_h, dec_h)


def _tag_layer(x, srcp, dstp, zrows, norm, W, b, out_dim, pad_dim):
    in_dim = x.shape[1]
    Wh = W.reshape(K + 1, in_dim, out_dim)
    if pad_dim != out_dim:
        pad = jnp.zeros((in_dim, pad_dim - out_dim), jnp.float32)
        blocks = [jnp.concatenate([Wh[i], pad], axis=1) for i in range(K + 1)]
    else:
        blocks = [Wh[i] for i in range(K + 1)]
    Wcat = jnp.concatenate(blocks, axis=1)
    T = _mm(x, Wcat)
    acc = T[:, K * pad_dim:(K + 1) * pad_dim]
    for i in range(K - 1, -1, -1):
        acc = _propagate(acc, srcp, dstp, zrows, norm)
        acc = acc + T[:, i * pad_dim:(i + 1) * pad_dim]
    return (acc[:, :out_dim] + b) if pad_dim != out_dim else (acc + b)


def kernel(X_input, edge_index, W1, b1, W2, b2, Wa, ba, Wd1, bd1, Wd2, bd2,
           Wd3, bd3, Wm, bm, Ws, bs, Wp, bp, mu):
    src, dst = edge_index[0], edge_index[1]
    padlen = EP - E
    srcp = jnp.concatenate(
        [src.astype(jnp.int32), jnp.full((padlen,), N, jnp.int32)]
    ).reshape(NW * CHUNKS_W, EB)
    dstp = jnp.concatenate(
        [dst.astype(jnp.int32), jnp.full((padlen,), N, jnp.int32)]
    ).reshape(NW * CHUNKS_W, EB)

    zrows128 = jnp.zeros((ZROWS, HID), jnp.float32)
    zrows16 = jnp.zeros((ZROWS, 16), jnp.float32)

    dparts = _sc_degree(dstp, zrows16)
    deg = dparts[0, :N, 0] + dparts[1, :N, 0]
    norm = jnp.where(deg > 0.0, deg, 1.0) ** -0.5

    enc_h = _tag_layer(X_input, srcp, dstp, zrows128, norm, W1, b1, HID, HID)
    z = _tag_layer(enc_h, srcp, dstp, zrows16, norm, W2, b2, LAT, 16)

    dec_h, q, _mean, _disp, _pi = _decoder_heads(
        z, Wa, ba, Wd1, bd1, Wd2, bd2, Wd3, bd3, Wm, bm, Ws, bs, Wp, bp, mu)
    A_out = _adj_out(dec_h)
    return (A_out, z, q, _mean, _disp, _pi)


# speedup vs baseline: 8.7643x; 2.0462x over previous
import functools
import jax
import jax.numpy as jnp
from jax import lax
from jax.experimental import pallas as pl
from jax.experimental.pallas import tpu as pltpu
from jax.experimental.pallas import tpu_sc as plsc

N = 10000
E = 160000
IN_DIM = 256
HID = 128
LAT = 15
ADJ_DIM = 32
K = 3
NCLUST = 10

NC = 2
NS = 16
NW = NC * NS
EB = 128
CHUNKS_W = 40
EP = NW * CHUNKS_W * EB
NP = 10240
ROWS_PER_SUB = NP // NS
ZROWS = 128

ROWS_B = 400


@functools.lru_cache(maxsize=None)
def _sc_segsum_kernel(D):
    NBUF = 2 if D >= 64 else 8
    mesh = plsc.VectorSubcoreMesh(core_axis_name="c", subcore_axis_name="s",
                                  num_cores=NC, num_subcores=NS)

    @functools.partial(
        pl.kernel,
        out_type=jax.ShapeDtypeStruct((NC, NP, D), jnp.float32),
        mesh=mesh,
        scratch_types=[
            pltpu.VMEM((CHUNKS_W, EB), jnp.int32),
            pltpu.VMEM((CHUNKS_W, EB), jnp.int32),
            pltpu.VMEM((NBUF, EB, D), jnp.float32),
            pltpu.VMEM_SHARED((NP, D), jnp.float32),
            pltpu.SemaphoreType.DMA((NBUF,)),
            pltpu.SemaphoreType.DMA((NBUF,)),
        ],
        compiler_params=pltpu.CompilerParams(use_tc_tiling_on_sc=False),
    )
    def seg_kernel(v_hbm, src_hbm, dst_hbm, z_hbm, out_hbm,
                   src_v, dst_v, rows, acc_sh, gsem, ssem):
        c = lax.axis_index("c")
        s = lax.axis_index("s")
        w = s * NC + c
        row0 = s * ROWS_PER_SUB
        nz = ROWS_PER_SUB // ZROWS
        for j in range(nz):
            pltpu.async_copy(z_hbm, acc_sh.at[pl.ds(row0 + j * ZROWS, ZROWS)],
                             gsem.at[j % NBUF])
        pltpu.async_copy(src_hbm.at[pl.ds(w * CHUNKS_W, CHUNKS_W)], src_v,
                         ssem.at[0])
        pltpu.async_copy(dst_hbm.at[pl.ds(w * CHUNKS_W, CHUNKS_W)], dst_v,
                         ssem.at[1])
        for j in range(nz):
            pltpu.make_async_copy(
                z_hbm, acc_sh.at[pl.ds(row0 + j * ZROWS, ZROWS)],
                gsem.at[j % NBUF]).wait()
        pltpu.make_async_copy(src_hbm.at[pl.ds(w * CHUNKS_W, CHUNKS_W)],
                              src_v, ssem.at[0]).wait()
        pltpu.make_async_copy(dst_hbm.at[pl.ds(w * CHUNKS_W, CHUNKS_W)],
                              dst_v, ssem.at[1]).wait()
        plsc.subcore_barrier()

        dummy = v_hbm.at[pl.ds(0, EB)]
        for b in range(NBUF):
            pltpu.async_copy(v_hbm.at[src_v.at[b]], rows.at[b], gsem.at[b])

        nblk = CHUNKS_W // NBUF

        @pl.loop(0, nblk)
        def _(k0):
            kb = k0 * NBUF
            for b in range(NBUF):
                pltpu.make_async_copy(dummy, rows.at[b], gsem.at[b]).wait()
                pltpu.async_copy(rows.at[b], acc_sh.at[dst_v.at[kb + b]],
                                 ssem.at[b], add=True)
            for b in range(NBUF):
                @pl.when(k0 < nblk - 1)
                def _():
                    pltpu.make_async_copy(dummy, rows.at[b], ssem.at[b]).wait()
                    pltpu.async_copy(v_hbm.at[src_v.at[kb + NBUF + b]],
                                     rows.at[b], gsem.at[b])

        for b in range(NBUF):
            pltpu.make_async_copy(dummy, rows.at[b], ssem.at[b]).wait()
        plsc.subcore_barrier()
        pltpu.sync_copy(acc_sh.at[pl.ds(row0, ROWS_PER_SUB)],
                        out_hbm.at[c].at[pl.ds(row0, ROWS_PER_SUB)])

    return seg_kernel


def _sc_segsum(v_pad, srcp, dstp, zrows):
    return _sc_segsum_kernel(v_pad.shape[1])(v_pad, srcp, dstp, zrows)


@functools.lru_cache(maxsize=None)
def _sc_degree_kernel():
    mesh = plsc.VectorSubcoreMesh(core_axis_name="c", subcore_axis_name="s",
                                  num_cores=NC, num_subcores=NS)

    @functools.partial(
        pl.kernel,
        out_type=jax.ShapeDtypeStruct((NC, NP, 16), jnp.float32),
        mesh=mesh,
        scratch_types=[
            pltpu.VMEM((CHUNKS_W, EB), jnp.int32),
            pltpu.VMEM((EB, 16), jnp.float32),
            pltpu.VMEM_SHARED((NP, 16), jnp.float32),
            pltpu.SemaphoreType.DMA,
        ],
        compiler_params=pltpu.CompilerParams(use_tc_tiling_on_sc=False),
    )
    def deg_kernel(dst_hbm, z_hbm, out_hbm, dst_v, ones_v, acc_sh, sem):
        c = lax.axis_index("c")
        s = lax.axis_index("s")
        w = s * NC + c
        row0 = s * ROWS_PER_SUB

        @pl.loop(0, EB)
        def _(r):
            ones_v.at[pl.ds(r, 1), :][...] = jnp.ones((1, 16), jnp.float32)

        for j in range(ROWS_PER_SUB // ZROWS):
            pltpu.sync_copy(z_hbm, acc_sh.at[pl.ds(row0 + j * ZROWS, ZROWS)])
        pltpu.sync_copy(dst_hbm.at[pl.ds(w * CHUNKS_W, CHUNKS_W)], dst_v)
        plsc.subcore_barrier()

        @pl.loop(0, CHUNKS_W)
        def _(k):
            pltpu.async_copy(ones_v, acc_sh.at[dst_v.at[k]], sem, add=True)

        @pl.loop(0, CHUNKS_W)
        def _(k):
            pltpu.make_async_copy(ones_v, acc_sh.at[dst_v.at[k]], sem).wait()

        plsc.subcore_barrier()
        pltpu.sync_copy(acc_sh.at[pl.ds(row0, ROWS_PER_SUB)],
                        out_hbm.at[c].at[pl.ds(row0, ROWS_PER_SUB)])

    return deg_kernel


def _sc_degree(dstp, zrows16):
    return _sc_degree_kernel()(dstp, zrows16)


def _propagate(v, srcp, dstp, zrows, norm):
    vpre = v * norm[:, None]
    v_pad = jnp.concatenate(
        [vpre, jnp.zeros((NP - N, v.shape[1]), jnp.float32)], axis=0)
    parts = _sc_segsum(v_pad, srcp, dstp, zrows)
    return (parts[0, :N] + parts[1, :N]) * norm[:, None]


def _mm_body(x_ref, w_ref, o_ref):
    o_ref[...] = jnp.dot(x_ref[...], w_ref[...],
                         preferred_element_type=jnp.float32)


def _mm(x, w):
    n, din = x.shape
    dout = w.shape[1]
    return pl.pallas_call(
        _mm_body,
        grid=(n // ROWS_B,),
        in_specs=[pl.BlockSpec((ROWS_B, din), lambda i: (i, 0)),
                  pl.BlockSpec((din, dout), lambda i: (0, 0))],
        out_specs=pl.BlockSpec((ROWS_B, dout), lambda i: (i, 0)),
        out_shape=jax.ShapeDtypeStruct((n, dout), jnp.float32),
    )(x, w)


def _heads_body(z_ref, wa_ref, ba_ref, wd1_ref, bd1_ref, wd2_ref, bd2_ref,
                wd3_ref, bd3_ref, wm_ref, bm_ref, ws_ref, bs_ref, wp_ref,
                bp_ref, mu_ref, dech_ref, q_ref, mean_ref, disp_ref, pi_ref):
    z = z_ref[...]
    f32 = jnp.float32
    dot = functools.partial(jnp.dot, preferred_element_type=f32)
    dech_ref[...] = dot(z, wa_ref[...]) + ba_ref[...]
    h = jax.nn.relu(dot(z, wd1_ref[...]) + bd1_ref[...])
    h = jax.nn.relu(dot(h, wd2_ref[...]) + bd2_ref[...])
    h = jax.nn.relu(dot(h, wd3_ref[...]) + bd3_ref[...])
    mean_ref[...] = jnp.clip(jnp.exp(dot(h, wm_ref[...]) + bm_ref[...]),
                             1e-5, 1e6)
    disp_ref[...] = jnp.clip(jax.nn.softplus(dot(h, ws_ref[...]) + bs_ref[...]),
                             1e-4, 1e4)
    pi_ref[...] = jax.nn.sigmoid(dot(h, wp_ref[...]) + bp_ref[...])
    mu = mu_ref[...]
    cols = []
    for j in range(NCLUST):
        d = z - mu[j][None, :]
        cols.append(jnp.sum(d * d, axis=1, keepdims=True))
    dist = jnp.concatenate(cols, axis=1)
    q = 1.0 / (1.0 + dist)
    q_ref[...] = q / jnp.sum(q, axis=1, keepdims=True)


def _adj_body(dech_blk_ref, dech_all_ref, out_ref):
    out_ref[...] = jax.nn.sigmoid(
        jnp.dot(dech_blk_ref[...], dech_all_ref[...].T,
                preferred_element_type=jnp.float32))


def _decoder_heads(z, Wa, ba, Wd1, bd1, Wd2, bd2, Wd3, bd3, Wm, bm, Ws, bs,
                   Wp, bp, mu):
    grid = (N // ROWS_B,)
    row_spec = lambda d: pl.BlockSpec((ROWS_B, d), lambda i: (i, 0))
    full = lambda a: pl.BlockSpec(a.shape, lambda i: tuple(0 for _ in a.shape))
    out_shapes = (
        jax.ShapeDtypeStruct((N, ADJ_DIM), jnp.float32),
        jax.ShapeDtypeStruct((N, NCLUST), jnp.float32),
        jax.ShapeDtypeStruct((N, IN_DIM), jnp.float32),
        jax.ShapeDtypeStruct((N, IN_DIM), jnp.float32),
        jax.ShapeDtypeStruct((N, IN_DIM), jnp.float32),
    )
    consts = (Wa, ba, Wd1, bd1, Wd2, bd2, Wd3, bd3, Wm, bm, Ws, bs, Wp, bp, mu)
    return pl.pallas_call(
        _heads_body,
        grid=grid,
        in_specs=[row_spec(LAT)] + [full(c) for c in consts],
        out_specs=(row_spec(ADJ_DIM), row_spec(NCLUST), row_spec(IN_DIM),
                   row_spec(IN_DIM), row_spec(IN_DIM)),
        out_shape=out_shapes,
    )(z, *consts)


def _adj_out(dec_h):
    grid = (N // ROWS_B,)
    return pl.pallas_call(
        _adj_body,
        grid=grid,
        in_specs=[pl.BlockSpec((ROWS_B, ADJ_DIM), lambda i: (i, 0)),
                  pl.BlockSpec((N, ADJ_DIM), lambda i: (0, 0))],
        out_specs=pl.BlockSpec((ROWS_B, N), lambda i: (i, 0)),
        out_shape=jax.ShapeDtypeStruct((N, N), jnp.float32),
    )(dec_h, dec_h)


def kernel(X_input, edge_index, W1, b1, W2, b2, Wa, ba, Wd1, bd1, Wd2, bd2,
           Wd3, bd3, Wm, bm, Ws, bs, Wp, bp, mu):
    src, dst = edge_index[0], edge_index[1]
    padlen = EP - E
    srcp = jnp.concatenate(
        [src.astype(jnp.int32), jnp.full((padlen,), N, jnp.int32)]
    ).reshape(NW * CHUNKS_W, EB)
    dstp = jnp.concatenate(
        [dst.astype(jnp.int32), jnp.full((padlen,), N, jnp.int32)]
    ).reshape(NW * CHUNKS_W, EB)

    zrows16 = jnp.zeros((ZROWS, 16), jnp.float32)

    dparts = _sc_degree(dstp, zrows16)
    deg = dparts[0, :N, 0] + dparts[1, :N, 0]
    norm = jnp.where(deg > 0.0, deg, 1.0) ** -0.5

    W1h = W1.reshape(K + 1, IN_DIM, HID)
    W2h = W2.reshape(K + 1, HID, LAT)
    KK = 2 * K
    Cs = {}
    for i in range(K + 1):
        for j in range(K + 1):
            kk = i + j
            prod = W1h[j] @ W2h[i]
            Cs[kk] = prod if kk not in Cs else Cs[kk] + prod
    colpad = jnp.zeros((IN_DIM, 16 - LAT), jnp.float32)
    Ccat = jnp.concatenate(
        sum(([Cs[kk], colpad] for kk in range(KK + 1)), []), axis=1)
    T = _mm(X_input, Ccat)
    cvec = [jnp.pad(b1 @ W2h[i], (0, 16 - LAT)) for i in range(K + 1)]

    def inject(kk):
        t = T[:, kk * 16:(kk + 1) * 16]
        return t + cvec[kk][None, :] if kk <= K else t

    acc = inject(KK)
    for kk in range(KK - 1, -1, -1):
        acc = _propagate(acc, srcp, dstp, zrows16, norm)
        acc = acc + inject(kk)
    z = acc[:, :LAT] + b2

    dec_h, q, _mean, _disp, _pi = _decoder_heads(
        z, Wa, ba, Wd1, bd1, Wd2, bd2, Wd3, bd3, Wm, bm, Ws, bs, Wp, bp, mu)
    A_out = _adj_out(dec_h)
    return (A_out, z, q, _mean, _disp, _pi)


# speedup vs baseline: 11.1315x; 1.2701x over previous
import functools
import jax
import jax.numpy as jnp
from jax import lax
from jax.experimental import pallas as pl
from jax.experimental.pallas import tpu as pltpu
from jax.experimental.pallas import tpu_sc as plsc

N = 10000
E = 160000
IN_DIM = 256
HID = 128
LAT = 15
ADJ_DIM = 32
K = 3
NCLUST = 10

NC = 2
NS = 16
NW = NC * NS
EB = 128
CHUNKS_W = 40
EP = NW * CHUNKS_W * EB
NP = 10240
ROWS_PER_SUB = NP // NS
ZROWS = 128

ROWS_B = 400


@functools.lru_cache(maxsize=None)
def _sc_segsum_kernel(D):
    NBUF = 2 if D >= 64 else 8
    mesh = plsc.VectorSubcoreMesh(core_axis_name="c", subcore_axis_name="s",
                                  num_cores=NC, num_subcores=NS)

    @functools.partial(
        pl.kernel,
        out_type=jax.ShapeDtypeStruct((NC, NP, D), jnp.float32),
        mesh=mesh,
        scratch_types=[
            pltpu.VMEM((CHUNKS_W, EB), jnp.int32),
            pltpu.VMEM((CHUNKS_W, EB), jnp.int32),
            pltpu.VMEM((NBUF, EB, D), jnp.float32),
            pltpu.VMEM_SHARED((NP, D), jnp.float32),
            pltpu.SemaphoreType.DMA((NBUF,)),
            pltpu.SemaphoreType.DMA((NBUF,)),
        ],
        compiler_params=pltpu.CompilerParams(use_tc_tiling_on_sc=False),
    )
    def seg_kernel(v_hbm, src_hbm, dst_hbm, z_hbm, out_hbm,
                   src_v, dst_v, rows, acc_sh, gsem, ssem):
        c = lax.axis_index("c")
        s = lax.axis_index("s")
        w = s * NC + c
        row0 = s * ROWS_PER_SUB
        nz = ROWS_PER_SUB // ZROWS
        for j in range(nz):
            pltpu.async_copy(z_hbm, acc_sh.at[pl.ds(row0 + j * ZROWS, ZROWS)],
                             gsem.at[j % NBUF])
        pltpu.async_copy(src_hbm.at[pl.ds(w * CHUNKS_W, CHUNKS_W)], src_v,
                         ssem.at[0])
        pltpu.async_copy(dst_hbm.at[pl.ds(w * CHUNKS_W, CHUNKS_W)], dst_v,
                         ssem.at[1])
        for j in range(nz):
            pltpu.make_async_copy(
                z_hbm, acc_sh.at[pl.ds(row0 + j * ZROWS, ZROWS)],
                gsem.at[j % NBUF]).wait()
        pltpu.make_async_copy(src_hbm.at[pl.ds(w * CHUNKS_W, CHUNKS_W)],
                              src_v, ssem.at[0]).wait()
        pltpu.make_async_copy(dst_hbm.at[pl.ds(w * CHUNKS_W, CHUNKS_W)],
                              dst_v, ssem.at[1]).wait()
        plsc.subcore_barrier()

        dummy = v_hbm.at[pl.ds(0, EB)]
        for b in range(NBUF):
            pltpu.async_copy(v_hbm.at[src_v.at[b]], rows.at[b], gsem.at[b])

        nblk = CHUNKS_W // NBUF

        @pl.loop(0, nblk)
        def _(k0):
            kb = k0 * NBUF
            for b in range(NBUF):
                pltpu.make_async_copy(dummy, rows.at[b], gsem.at[b]).wait()
                pltpu.async_copy(rows.at[b], acc_sh.at[dst_v.at[kb + b]],
                                 ssem.at[b], add=True)
            for b in range(NBUF):
                @pl.when(k0 < nblk - 1)
                def _():
                    pltpu.make_async_copy(dummy, rows.at[b], ssem.at[b]).wait()
                    pltpu.async_copy(v_hbm.at[src_v.at[kb + NBUF + b]],
                                     rows.at[b], gsem.at[b])

        for b in range(NBUF):
            pltpu.make_async_copy(dummy, rows.at[b], ssem.at[b]).wait()
        plsc.subcore_barrier()
        pltpu.sync_copy(acc_sh.at[pl.ds(row0, ROWS_PER_SUB)],
                        out_hbm.at[c].at[pl.ds(row0, ROWS_PER_SUB)])

    return seg_kernel


def _sc_segsum(v_pad, srcp, dstp, zrows):
    return _sc_segsum_kernel(v_pad.shape[1])(v_pad, srcp, dstp, zrows)


@functools.lru_cache(maxsize=None)
def _sc_degree_kernel():
    mesh = plsc.VectorSubcoreMesh(core_axis_name="c", subcore_axis_name="s",
                                  num_cores=NC, num_subcores=NS)

    @functools.partial(
        pl.kernel,
        out_type=jax.ShapeDtypeStruct((NC, NP, 16), jnp.float32),
        mesh=mesh,
        scratch_types=[
            pltpu.VMEM((CHUNKS_W, EB), jnp.int32),
            pltpu.VMEM((EB, 16), jnp.float32),
            pltpu.VMEM_SHARED((NP, 16), jnp.float32),
            pltpu.SemaphoreType.DMA,
        ],
        compiler_params=pltpu.CompilerParams(use_tc_tiling_on_sc=False),
    )
    def deg_kernel(dst_hbm, z_hbm, out_hbm, dst_v, ones_v, acc_sh, sem):
        c = lax.axis_index("c")
        s = lax.axis_index("s")
        w = s * NC + c
        row0 = s * ROWS_PER_SUB

        @pl.loop(0, EB)
        def _(r):
            ones_v.at[pl.ds(r, 1), :][...] = jnp.ones((1, 16), jnp.float32)

        for j in range(ROWS_PER_SUB // ZROWS):
            pltpu.sync_copy(z_hbm, acc_sh.at[pl.ds(row0 + j * ZROWS, ZROWS)])
        pltpu.sync_copy(dst_hbm.at[pl.ds(w * CHUNKS_W, CHUNKS_W)], dst_v)
        plsc.subcore_barrier()

        @pl.loop(0, CHUNKS_W)
        def _(k):
            pltpu.async_copy(ones_v, acc_sh.at[dst_v.at[k]], sem, add=True)

        @pl.loop(0, CHUNKS_W)
        def _(k):
            pltpu.make_async_copy(ones_v, acc_sh.at[dst_v.at[k]], sem).wait()

        plsc.subcore_barrier()
        pltpu.sync_copy(acc_sh.at[pl.ds(row0, ROWS_PER_SUB)],
                        out_hbm.at[c].at[pl.ds(row0, ROWS_PER_SUB)])

    return deg_kernel


def _sc_degree(dstp, zrows16):
    return _sc_degree_kernel()(dstp, zrows16)


def _propagate(v, srcp, dstp, zrows, norm):
    vpre = v * norm[:, None]
    v_pad = jnp.concatenate(
        [vpre, jnp.zeros((NP - N, v.shape[1]), jnp.float32)], axis=0)
    parts = _sc_segsum(v_pad, srcp, dstp, zrows)
    return (parts[0, :N] + parts[1, :N]) * norm[:, None]


def _mm_body(x_ref, w_ref, o_ref):
    o_ref[...] = jnp.dot(x_ref[...], w_ref[...],
                         preferred_element_type=jnp.float32)


def _mm(x, w):
    n, din = x.shape
    dout = w.shape[1]
    return pl.pallas_call(
        _mm_body,
        grid=(n // ROWS_B,),
        in_specs=[pl.BlockSpec((ROWS_B, din), lambda i: (i, 0)),
                  pl.BlockSpec((din, dout), lambda i: (0, 0))],
        out_specs=pl.BlockSpec((ROWS_B, dout), lambda i: (i, 0)),
        out_shape=jax.ShapeDtypeStruct((n, dout), jnp.float32),
    )(x, w)


def _heads_body(z_ref, wa_ref, ba_ref, wd1_ref, bd1_ref, wd2_ref, bd2_ref,
                wd3_ref, bd3_ref, wm_ref, bm_ref, ws_ref, bs_ref, wp_ref,
                bp_ref, mu_ref, dech_ref, q_ref, mean_ref, disp_ref, pi_ref):
    z = z_ref[...]
    f32 = jnp.float32
    dot = functools.partial(jnp.dot, preferred_element_type=f32)
    dech_ref[...] = dot(z, wa_ref[...]) + ba_ref[...]
    h = jax.nn.relu(dot(z, wd1_ref[...]) + bd1_ref[...])
    h = jax.nn.relu(dot(h, wd2_ref[...]) + bd2_ref[...])
    h = jax.nn.relu(dot(h, wd3_ref[...]) + bd3_ref[...])
    mean_ref[...] = jnp.clip(jnp.exp(dot(h, wm_ref[...]) + bm_ref[...]),
                             1e-5, 1e6)
    disp_ref[...] = jnp.clip(jax.nn.softplus(dot(h, ws_ref[...]) + bs_ref[...]),
                             1e-4, 1e4)
    pi_ref[...] = jax.nn.sigmoid(dot(h, wp_ref[...]) + bp_ref[...])
    mu = mu_ref[...]
    cols = []
    for j in range(NCLUST):
        d = z - mu[j][None, :]
        cols.append(jnp.sum(d * d, axis=1, keepdims=True))
    dist = jnp.concatenate(cols, axis=1)
    q = 1.0 / (1.0 + dist)
    q_ref[...] = q / jnp.sum(q, axis=1, keepdims=True)


def _adj_body(dech_blk_ref, dech_all_ref, out_ref):
    out_ref[...] = jax.nn.sigmoid(
        jnp.dot(dech_blk_ref[...], dech_all_ref[...].T,
                preferred_element_type=jnp.float32))


def _decoder_heads(z, Wa, ba, Wd1, bd1, Wd2, bd2, Wd3, bd3, Wm, bm, Ws, bs,
                   Wp, bp, mu):
    grid = (N // ROWS_B,)
    row_spec = lambda d: pl.BlockSpec((ROWS_B, d), lambda i: (i, 0))
    full = lambda a: pl.BlockSpec(a.shape, lambda i: tuple(0 for _ in a.shape))
    out_shapes = (
        jax.ShapeDtypeStruct((N, ADJ_DIM), jnp.float32),
        jax.ShapeDtypeStruct((N, NCLUST), jnp.float32),
        jax.ShapeDtypeStruct((N, IN_DIM), jnp.float32),
        jax.ShapeDtypeStruct((N, IN_DIM), jnp.float32),
        jax.ShapeDtypeStruct((N, IN_DIM), jnp.float32),
    )
    consts = (Wa, ba, Wd1, bd1, Wd2, bd2, Wd3, bd3, Wm, bm, Ws, bs, Wp, bp, mu)
    return pl.pallas_call(
        _heads_body,
        grid=grid,
        in_specs=[row_spec(LAT)] + [full(c) for c in consts],
        out_specs=(row_spec(ADJ_DIM), row_spec(NCLUST), row_spec(IN_DIM),
                   row_spec(IN_DIM), row_spec(IN_DIM)),
        out_shape=out_shapes,
    )(z, *consts)


def _adj_out(dec_h):
    grid = (N // ROWS_B,)
    return pl.pallas_call(
        _adj_body,
        grid=grid,
        in_specs=[pl.BlockSpec((ROWS_B, ADJ_DIM), lambda i: (i, 0)),
                  pl.BlockSpec((N, ADJ_DIM), lambda i: (0, 0))],
        out_specs=pl.BlockSpec((ROWS_B, N), lambda i: (i, 0)),
        out_shape=jax.ShapeDtypeStruct((N, N), jnp.float32),
    )(dec_h, dec_h)


def kernel(X_input, edge_index, W1, b1, W2, b2, Wa, ba, Wd1, bd1, Wd2, bd2,
           Wd3, bd3, Wm, bm, Ws, bs, Wp, bp, mu):
    src, dst = edge_index[0], edge_index[1]
    padlen = EP - E
    padidx = N + (jnp.arange(padlen, dtype=jnp.int32) % (NP - N))
    srcp = jnp.concatenate(
        [src.astype(jnp.int32), padidx]).reshape(NW * CHUNKS_W, EB)
    dstp = jnp.concatenate(
        [dst.astype(jnp.int32), padidx]).reshape(NW * CHUNKS_W, EB)

    zrows16 = jnp.zeros((ZROWS, 16), jnp.float32)

    dparts = _sc_degree(dstp, zrows16)
    deg = dparts[0, :N, 0] + dparts[1, :N, 0]
    norm = jnp.where(deg > 0.0, deg, 1.0) ** -0.5

    W1h = W1.reshape(K + 1, IN_DIM, HID)
    W2h = W2.reshape(K + 1, HID, LAT)
    KK = 2 * K
    Cs = {}
    for i in range(K + 1):
        for j in range(K + 1):
            kk = i + j
            prod = W1h[j] @ W2h[i]
            Cs[kk] = prod if kk not in Cs else Cs[kk] + prod
    colpad = jnp.zeros((IN_DIM, 16 - LAT), jnp.float32)
    Ccat = jnp.concatenate(
        sum(([Cs[kk], colpad] for kk in range(KK + 1)), []), axis=1)
    T = _mm(X_input, Ccat)
    cvec = [jnp.pad(b1 @ W2h[i], (0, 16 - LAT)) for i in range(K + 1)]

    def inject(kk):
        t = T[:, kk * 16:(kk + 1) * 16]
        return t + cvec[kk][None, :] if kk <= K else t

    acc = inject(KK)
    for kk in range(KK - 1, -1, -1):
        acc = _propagate(acc, srcp, dstp, zrows16, norm)
        acc = acc + inject(kk)
    z = acc[:, :LAT] + b2

    dec_h, q, _mean, _disp, _pi = _decoder_heads(
        z, Wa, ba, Wd1, bd1, Wd2, bd2, Wd3, bd3, Wm, bm, Ws, bs, Wp, bp, mu)
    A_out = _adj_out(dec_h)
    return (A_out, z, q, _mean, _disp, _pi)


# speedup vs baseline: 11.5575x; 1.0383x over previous
import functools
import jax
import jax.numpy as jnp
from jax import lax
from jax.experimental import pallas as pl
from jax.experimental.pallas import tpu as pltpu
from jax.experimental.pallas import tpu_sc as plsc

N = 10000
E = 160000
IN_DIM = 256
HID = 128
LAT = 15
ADJ_DIM = 32
K = 3
NCLUST = 10

NC = 2
NS = 16
NW = NC * NS
EB = 128
CHUNKS_W = 40
EP = NW * CHUNKS_W * EB
NP = 10240
ROWS_PER_SUB = NP // NS
ZROWS = 128

ROWS_B = 400


@functools.lru_cache(maxsize=None)
def _sc_segsum_kernel(D):
    NBUF = 2 if D >= 64 else 8
    mesh = plsc.VectorSubcoreMesh(core_axis_name="c", subcore_axis_name="s",
                                  num_cores=NC, num_subcores=NS)

    @functools.partial(
        pl.kernel,
        out_type=jax.ShapeDtypeStruct((NC, NP, D), jnp.float32),
        mesh=mesh,
        scratch_types=[
            pltpu.VMEM((CHUNKS_W, EB), jnp.int32),
            pltpu.VMEM((CHUNKS_W, EB), jnp.int32),
            pltpu.VMEM((NBUF, EB, D), jnp.float32),
            pltpu.VMEM_SHARED((NP, D), jnp.float32),
            pltpu.SemaphoreType.DMA((NBUF,)),
            pltpu.SemaphoreType.DMA((NBUF,)),
        ],
        compiler_params=pltpu.CompilerParams(use_tc_tiling_on_sc=False),
    )
    def seg_kernel(v_hbm, src_hbm, dst_hbm, z_hbm, out_hbm,
                   src_v, dst_v, rows, acc_sh, gsem, ssem):
        c = lax.axis_index("c")
        s = lax.axis_index("s")
        w = s * NC + c
        row0 = s * ROWS_PER_SUB
        nz = ROWS_PER_SUB // ZROWS
        for j in range(nz):
            pltpu.async_copy(z_hbm, acc_sh.at[pl.ds(row0 + j * ZROWS, ZROWS)],
                             gsem.at[j % NBUF])
        pltpu.async_copy(src_hbm.at[pl.ds(w * CHUNKS_W, CHUNKS_W)], src_v,
                         ssem.at[0])
        pltpu.async_copy(dst_hbm.at[pl.ds(w * CHUNKS_W, CHUNKS_W)], dst_v,
                         ssem.at[1])
        for j in range(nz):
            pltpu.make_async_copy(
                z_hbm, acc_sh.at[pl.ds(row0 + j * ZROWS, ZROWS)],
                gsem.at[j % NBUF]).wait()
        pltpu.make_async_copy(src_hbm.at[pl.ds(w * CHUNKS_W, CHUNKS_W)],
                              src_v, ssem.at[0]).wait()
        pltpu.make_async_copy(dst_hbm.at[pl.ds(w * CHUNKS_W, CHUNKS_W)],
                              dst_v, ssem.at[1]).wait()
        plsc.subcore_barrier()

        dummy = v_hbm.at[pl.ds(0, EB)]
        for b in range(NBUF):
            pltpu.async_copy(v_hbm.at[src_v.at[b]], rows.at[b], gsem.at[b])

        nblk = CHUNKS_W // NBUF

        @pl.loop(0, nblk)
        def _(k0):
            kb = k0 * NBUF
            for b in range(NBUF):
                pltpu.make_async_copy(dummy, rows.at[b], gsem.at[b]).wait()
                pltpu.async_copy(rows.at[b], acc_sh.at[dst_v.at[kb + b]],
                                 ssem.at[b], add=True)
            for b in range(NBUF):
                @pl.when(k0 < nblk - 1)
                def _():
                    pltpu.make_async_copy(dummy, rows.at[b], ssem.at[b]).wait()
                    pltpu.async_copy(v_hbm.at[src_v.at[kb + NBUF + b]],
                                     rows.at[b], gsem.at[b])

        for b in range(NBUF):
            pltpu.make_async_copy(dummy, rows.at[b], ssem.at[b]).wait()
        plsc.subcore_barrier()
        pltpu.sync_copy(acc_sh.at[pl.ds(row0, ROWS_PER_SUB)],
                        out_hbm.at[c].at[pl.ds(row0, ROWS_PER_SUB)])

    return seg_kernel


def _sc_segsum(v_pad, srcp, dstp, zrows):
    return _sc_segsum_kernel(v_pad.shape[1])(v_pad, srcp, dstp, zrows)


@functools.lru_cache(maxsize=None)
def _sc_degree_kernel():
    mesh = plsc.VectorSubcoreMesh(core_axis_name="c", subcore_axis_name="s",
                                  num_cores=NC, num_subcores=NS)

    @functools.partial(
        pl.kernel,
        out_type=jax.ShapeDtypeStruct((NC, NP, 16), jnp.float32),
        mesh=mesh,
        scratch_types=[
            pltpu.VMEM((CHUNKS_W, EB), jnp.int32),
            pltpu.VMEM((EB, 16), jnp.float32),
            pltpu.VMEM_SHARED((NP, 16), jnp.float32),
            pltpu.SemaphoreType.DMA,
        ],
        compiler_params=pltpu.CompilerParams(use_tc_tiling_on_sc=False),
    )
    def deg_kernel(dst_hbm, z_hbm, out_hbm, dst_v, ones_v, acc_sh, sem):
        c = lax.axis_index("c")
        s = lax.axis_index("s")
        w = s * NC + c
        row0 = s * ROWS_PER_SUB

        @pl.loop(0, EB)
        def _(r):
            ones_v.at[pl.ds(r, 1), :][...] = jnp.ones((1, 16), jnp.float32)

        for j in range(ROWS_PER_SUB // ZROWS):
            pltpu.sync_copy(z_hbm, acc_sh.at[pl.ds(row0 + j * ZROWS, ZROWS)])
        pltpu.sync_copy(dst_hbm.at[pl.ds(w * CHUNKS_W, CHUNKS_W)], dst_v)
        plsc.subcore_barrier()

        @pl.loop(0, CHUNKS_W)
        def _(k):
            pltpu.async_copy(ones_v, acc_sh.at[dst_v.at[k]], sem, add=True)

        @pl.loop(0, CHUNKS_W)
        def _(k):
            pltpu.make_async_copy(ones_v, acc_sh.at[dst_v.at[k]], sem).wait()

        plsc.subcore_barrier()
        pltpu.sync_copy(acc_sh.at[pl.ds(row0, ROWS_PER_SUB)],
                        out_hbm.at[c].at[pl.ds(row0, ROWS_PER_SUB)])

    return deg_kernel


def _sc_degree(dstp, zrows16):
    return _sc_degree_kernel()(dstp, zrows16)


def _propagate(v, srcp, dstp, zrows, norm):
    vpre = v * norm[:, None]
    v_pad = jnp.concatenate(
        [vpre, jnp.zeros((NP - N, v.shape[1]), jnp.float32)], axis=0)
    parts = _sc_segsum(v_pad, srcp, dstp, zrows)
    return (parts[0, :N] + parts[1, :N]) * norm[:, None]


def _mm_body(x_ref, w_ref, o_ref):
    o_ref[...] = jnp.dot(x_ref[...], w_ref[...],
                         preferred_element_type=jnp.float32)


def _mm(x, w):
    n, din = x.shape
    dout = w.shape[1]
    return pl.pallas_call(
        _mm_body,
        grid=(n // ROWS_B,),
        in_specs=[pl.BlockSpec((ROWS_B, din), lambda i: (i, 0)),
                  pl.BlockSpec((din, dout), lambda i: (0, 0))],
        out_specs=pl.BlockSpec((ROWS_B, dout), lambda i: (i, 0)),
        out_shape=jax.ShapeDtypeStruct((n, dout), jnp.float32),
    )(x, w)


def _heads_body(z_ref, wa_ref, ba_ref, wd1_ref, bd1_ref, wd2_ref, bd2_ref,
                wd3_ref, bd3_ref, wm_ref, bm_ref, ws_ref, bs_ref, wp_ref,
                bp_ref, mu_ref, dech_ref, q_ref, mean_ref, disp_ref, pi_ref):
    z = z_ref[...]
    f32 = jnp.float32
    dot = functools.partial(jnp.dot, preferred_element_type=f32)
    dech_ref[...] = dot(z, wa_ref[...]) + ba_ref[...]
    h = jax.nn.relu(dot(z, wd1_ref[...]) + bd1_ref[...])
    h = jax.nn.relu(dot(h, wd2_ref[...]) + bd2_ref[...])
    h = jax.nn.relu(dot(h, wd3_ref[...]) + bd3_ref[...])
    mean_ref[...] = jnp.clip(jnp.exp(dot(h, wm_ref[...]) + bm_ref[...]),
                             1e-5, 1e6)
    disp_ref[...] = jnp.clip(jax.nn.softplus(dot(h, ws_ref[...]) + bs_ref[...]),
                             1e-4, 1e4)
    pi_ref[...] = jax.nn.sigmoid(dot(h, wp_ref[...]) + bp_ref[...])
    mu = mu_ref[...]
    cols = []
    for j in range(NCLUST):
        d = z - mu[j][None, :]
        cols.append(jnp.sum(d * d, axis=1, keepdims=True))
    dist = jnp.concatenate(cols, axis=1)
    q = 1.0 / (1.0 + dist)
    q_ref[...] = q / jnp.sum(q, axis=1, keepdims=True)


def _adj_body(dech_blk_ref, dech_all_ref, out_ref):
    s = jnp.dot(dech_blk_ref[...], dech_all_ref[...].T,
                preferred_element_type=jnp.float32)
    out_ref[...] = 0.5 + 0.5 * jnp.tanh(0.5 * s)


def _decoder_heads(z, Wa, ba, Wd1, bd1, Wd2, bd2, Wd3, bd3, Wm, bm, Ws, bs,
                   Wp, bp, mu):
    grid = (N // ROWS_B,)
    row_spec = lambda d: pl.BlockSpec((ROWS_B, d), lambda i: (i, 0))
    full = lambda a: pl.BlockSpec(a.shape, lambda i: tuple(0 for _ in a.shape))
    out_shapes = (
        jax.ShapeDtypeStruct((N, ADJ_DIM), jnp.float32),
        jax.ShapeDtypeStruct((N, NCLUST), jnp.float32),
        jax.ShapeDtypeStruct((N, IN_DIM), jnp.float32),
        jax.ShapeDtypeStruct((N, IN_DIM), jnp.float32),
        jax.ShapeDtypeStruct((N, IN_DIM), jnp.float32),
    )
    consts = (Wa, ba, Wd1, bd1, Wd2, bd2, Wd3, bd3, Wm, bm, Ws, bs, Wp, bp, mu)
    return pl.pallas_call(
        _heads_body,
        grid=grid,
        in_specs=[row_spec(LAT)] + [full(c) for c in consts],
        out_specs=(row_spec(ADJ_DIM), row_spec(NCLUST), row_spec(IN_DIM),
                   row_spec(IN_DIM), row_spec(IN_DIM)),
        out_shape=out_shapes,
    )(z, *consts)


def _adj_out(dec_h):
    grid = (N // ROWS_B,)
    return pl.pallas_call(
        _adj_body,
        grid=grid,
        in_specs=[pl.BlockSpec((ROWS_B, ADJ_DIM), lambda i: (i, 0)),
                  pl.BlockSpec((N, ADJ_DIM), lambda i: (0, 0))],
        out_specs=pl.BlockSpec((ROWS_B, N), lambda i: (i, 0)),
        out_shape=jax.ShapeDtypeStruct((N, N), jnp.float32),
    )(dec_h, dec_h)


def kernel(X_input, edge_index, W1, b1, W2, b2, Wa, ba, Wd1, bd1, Wd2, bd2,
           Wd3, bd3, Wm, bm, Ws, bs, Wp, bp, mu):
    src, dst = edge_index[0], edge_index[1]
    padlen = EP - E
    padidx = N + (jnp.arange(padlen, dtype=jnp.int32) % (NP - N))
    srcp = jnp.concatenate(
        [src.astype(jnp.int32), padidx]).reshape(NW * CHUNKS_W, EB)
    dstp = jnp.concatenate(
        [dst.astype(jnp.int32), padidx]).reshape(NW * CHUNKS_W, EB)

    zrows16 = jnp.zeros((ZROWS, 16), jnp.float32)

    dparts = _sc_degree(dstp, zrows16)
    deg = dparts[0, :N, 0] + dparts[1, :N, 0]
    norm = jnp.where(deg > 0.0, deg, 1.0) ** -0.5

    W1h = W1.reshape(K + 1, IN_DIM, HID)
    W2h = W2.reshape(K + 1, HID, LAT)
    KK = 2 * K
    Cs = {}
    for i in range(K + 1):
        for j in range(K + 1):
            kk = i + j
            prod = W1h[j] @ W2h[i]
            Cs[kk] = prod if kk not in Cs else Cs[kk] + prod
    colpad = jnp.zeros((IN_DIM, 16 - LAT), jnp.float32)
    Ccat = jnp.concatenate(
        sum(([Cs[kk], colpad] for kk in range(KK + 1)), []), axis=1)
    T = _mm(X_input, Ccat)
    cvec = [jnp.pad(b1 @ W2h[i], (0, 16 - LAT)) for i in range(K + 1)]

    def inject(kk):
        t = T[:, kk * 16:(kk + 1) * 16]
        return t + cvec[kk][None, :] if kk <= K else t

    acc = inject(KK)
    for kk in range(KK - 1, -1, -1):
        acc = _propagate(acc, srcp, dstp, zrows16, norm)
        acc = acc + inject(kk)
    z = acc[:, :LAT] + b2

    dec_h, q, _mean, _disp, _pi = _decoder_heads(
        z, Wa, ba, Wd1, bd1, Wd2, bd2, Wd3, bd3, Wm, bm, Ws, bs, Wp, bp, mu)
    A_out = _adj_out(dec_h)
    return (A_out, z, q, _mean, _disp, _pi)


# speedup vs baseline: 11.6536x; 1.0083x over previous
import functools
import jax
import jax.numpy as jnp
from jax import lax
from jax.experimental import pallas as pl
from jax.experimental.pallas import tpu as pltpu
from jax.experimental.pallas import tpu_sc as plsc

N = 10000
E = 160000
IN_DIM = 256
HID = 128
LAT = 15
ADJ_DIM = 32
K = 3
NCLUST = 10

NC = 2
NS = 16
NW = NC * NS
EB = 128
CHUNKS_W = 40
EP = NW * CHUNKS_W * EB
NP = 10240
ROWS_PER_SUB = NP // NS
ZROWS = 128

ROWS_B = 400


@functools.lru_cache(maxsize=None)
def _sc_segsum_kernel(D):
    NBUF = 2 if D >= 64 else 8
    mesh = plsc.VectorSubcoreMesh(core_axis_name="c", subcore_axis_name="s",
                                  num_cores=NC, num_subcores=NS)

    @functools.partial(
        pl.kernel,
        out_type=jax.ShapeDtypeStruct((NC, NP, D), jnp.float32),
        mesh=mesh,
        scratch_types=[
            pltpu.VMEM((CHUNKS_W, EB), jnp.int32),
            pltpu.VMEM((CHUNKS_W, EB), jnp.int32),
            pltpu.VMEM((NBUF, EB, D), jnp.float32),
            pltpu.VMEM_SHARED((NP, D), jnp.float32),
            pltpu.SemaphoreType.DMA((NBUF,)),
            pltpu.SemaphoreType.DMA((NBUF,)),
        ],
        compiler_params=pltpu.CompilerParams(use_tc_tiling_on_sc=False),
    )
    def seg_kernel(v_hbm, src_hbm, dst_hbm, z_hbm, out_hbm,
                   src_v, dst_v, rows, acc_sh, gsem, ssem):
        c = lax.axis_index("c")
        s = lax.axis_index("s")
        w = s * NC + c
        row0 = s * ROWS_PER_SUB
        nz = ROWS_PER_SUB // ZROWS
        for j in range(nz):
            pltpu.async_copy(z_hbm, acc_sh.at[pl.ds(row0 + j * ZROWS, ZROWS)],
                             gsem.at[j % NBUF])
        pltpu.async_copy(src_hbm.at[pl.ds(w * CHUNKS_W, CHUNKS_W)], src_v,
                         ssem.at[0])
        pltpu.async_copy(dst_hbm.at[pl.ds(w * CHUNKS_W, CHUNKS_W)], dst_v,
                         ssem.at[1])
        for j in range(nz):
            pltpu.make_async_copy(
                z_hbm, acc_sh.at[pl.ds(row0 + j * ZROWS, ZROWS)],
                gsem.at[j % NBUF]).wait()
        pltpu.make_async_copy(src_hbm.at[pl.ds(w * CHUNKS_W, CHUNKS_W)],
                              src_v, ssem.at[0]).wait()
        pltpu.make_async_copy(dst_hbm.at[pl.ds(w * CHUNKS_W, CHUNKS_W)],
                              dst_v, ssem.at[1]).wait()
        plsc.subcore_barrier()

        dummy = v_hbm.at[pl.ds(0, EB)]
        for b in range(NBUF):
            pltpu.async_copy(v_hbm.at[src_v.at[b]], rows.at[b], gsem.at[b])

        nblk = CHUNKS_W // NBUF

        @pl.loop(0, nblk)
        def _(k0):
            kb = k0 * NBUF
            for b in range(NBUF):
                pltpu.make_async_copy(dummy, rows.at[b], gsem.at[b]).wait()
                pltpu.async_copy(rows.at[b], acc_sh.at[dst_v.at[kb + b]],
                                 ssem.at[b], add=True)
            for b in range(NBUF):
                @pl.when(k0 < nblk - 1)
                def _():
                    pltpu.make_async_copy(dummy, rows.at[b], ssem.at[b]).wait()
                    pltpu.async_copy(v_hbm.at[src_v.at[kb + NBUF + b]],
                                     rows.at[b], gsem.at[b])

        for b in range(NBUF):
            pltpu.make_async_copy(dummy, rows.at[b], ssem.at[b]).wait()
        plsc.subcore_barrier()
        pltpu.sync_copy(acc_sh.at[pl.ds(row0, ROWS_PER_SUB)],
                        out_hbm.at[c].at[pl.ds(row0, ROWS_PER_SUB)])

    return seg_kernel


def _sc_segsum(v_pad, srcp, dstp, zrows):
    return _sc_segsum_kernel(v_pad.shape[1])(v_pad, srcp, dstp, zrows)


@functools.lru_cache(maxsize=None)
def _sc_degree_kernel():
    mesh = plsc.VectorSubcoreMesh(core_axis_name="c", subcore_axis_name="s",
                                  num_cores=NC, num_subcores=NS)

    @functools.partial(
        pl.kernel,
        out_type=jax.ShapeDtypeStruct((NC, NP, 16), jnp.float32),
        mesh=mesh,
        scratch_types=[
            pltpu.VMEM((CHUNKS_W, EB), jnp.int32),
            pltpu.VMEM((EB, 16), jnp.float32),
            pltpu.VMEM_SHARED((NP, 16), jnp.float32),
            pltpu.SemaphoreType.DMA,
        ],
        compiler_params=pltpu.CompilerParams(use_tc_tiling_on_sc=False),
    )
    def deg_kernel(dst_hbm, z_hbm, out_hbm, dst_v, ones_v, acc_sh, sem):
        c = lax.axis_index("c")
        s = lax.axis_index("s")
        w = s * NC + c
        row0 = s * ROWS_PER_SUB

        @pl.loop(0, EB)
        def _(r):
            ones_v.at[pl.ds(r, 1), :][...] = jnp.ones((1, 16), jnp.float32)

        for j in range(ROWS_PER_SUB // ZROWS):
            pltpu.sync_copy(z_hbm, acc_sh.at[pl.ds(row0 + j * ZROWS, ZROWS)])
        pltpu.sync_copy(dst_hbm.at[pl.ds(w * CHUNKS_W, CHUNKS_W)], dst_v)
        plsc.subcore_barrier()

        @pl.loop(0, CHUNKS_W)
        def _(k):
            pltpu.async_copy(ones_v, acc_sh.at[dst_v.at[k]], sem, add=True)

        @pl.loop(0, CHUNKS_W)
        def _(k):
            pltpu.make_async_copy(ones_v, acc_sh.at[dst_v.at[k]], sem).wait()

        plsc.subcore_barrier()
        pltpu.sync_copy(acc_sh.at[pl.ds(row0, ROWS_PER_SUB)],
                        out_hbm.at[c].at[pl.ds(row0, ROWS_PER_SUB)])

    return deg_kernel


def _sc_degree(dstp, zrows16):
    return _sc_degree_kernel()(dstp, zrows16)


def _mm_body(x_ref, w_ref, o_ref):
    o_ref[...] = jnp.dot(x_ref[...], w_ref[...],
                         preferred_element_type=jnp.float32)


def _mm(x, w):
    n, din = x.shape
    dout = w.shape[1]
    return pl.pallas_call(
        _mm_body,
        grid=(n // ROWS_B,),
        in_specs=[pl.BlockSpec((ROWS_B, din), lambda i: (i, 0)),
                  pl.BlockSpec((din, dout), lambda i: (0, 0))],
        out_specs=pl.BlockSpec((ROWS_B, dout), lambda i: (i, 0)),
        out_shape=jax.ShapeDtypeStruct((n, dout), jnp.float32),
    )(x, w)


def _heads_body(z_ref, wa_ref, ba_ref, wd1_ref, bd1_ref, wd2_ref, bd2_ref,
                wd3_ref, bd3_ref, wm_ref, bm_ref, ws_ref, bs_ref, wp_ref,
                bp_ref, mu_ref, dech_ref, q_ref, mean_ref, disp_ref, pi_ref):
    z = z_ref[...]
    f32 = jnp.float32
    dot = functools.partial(jnp.dot, preferred_element_type=f32)
    dech_ref[...] = dot(z, wa_ref[...]) + ba_ref[...]
    h = jax.nn.relu(dot(z, wd1_ref[...]) + bd1_ref[...])
    h = jax.nn.relu(dot(h, wd2_ref[...]) + bd2_ref[...])
    h = jax.nn.relu(dot(h, wd3_ref[...]) + bd3_ref[...])
    mean_ref[...] = jnp.clip(jnp.exp(dot(h, wm_ref[...]) + bm_ref[...]),
                             1e-5, 1e6)
    disp_ref[...] = jnp.clip(jax.nn.softplus(dot(h, ws_ref[...]) + bs_ref[...]),
                             1e-4, 1e4)
    pi_ref[...] = jax.nn.sigmoid(dot(h, wp_ref[...]) + bp_ref[...])
    mu = mu_ref[...]
    cols = []
    for j in range(NCLUST):
        d = z - mu[j][None, :]
        cols.append(jnp.sum(d * d, axis=1, keepdims=True))
    dist = jnp.concatenate(cols, axis=1)
    q = 1.0 / (1.0 + dist)
    q_ref[...] = q / jnp.sum(q, axis=1, keepdims=True)


def _adj_body(dech_blk_ref, dech_all_ref, out_ref):
    s = jnp.dot(dech_blk_ref[...], dech_all_ref[...].T,
                preferred_element_type=jnp.float32)
    out_ref[...] = 0.5 + 0.5 * jnp.tanh(0.5 * s)


def _decoder_heads(z, Wa, ba, Wd1, bd1, Wd2, bd2, Wd3, bd3, Wm, bm, Ws, bs,
                   Wp, bp, mu):
    grid = (N // ROWS_B,)
    row_spec = lambda d: pl.BlockSpec((ROWS_B, d), lambda i: (i, 0))
    full = lambda a: pl.BlockSpec(a.shape, lambda i: tuple(0 for _ in a.shape))
    out_shapes = (
        jax.ShapeDtypeStruct((N, ADJ_DIM), jnp.float32),
        jax.ShapeDtypeStruct((N, NCLUST), jnp.float32),
        jax.ShapeDtypeStruct((N, IN_DIM), jnp.float32),
        jax.ShapeDtypeStruct((N, IN_DIM), jnp.float32),
        jax.ShapeDtypeStruct((N, IN_DIM), jnp.float32),
    )
    consts = (Wa, ba, Wd1, bd1, Wd2, bd2, Wd3, bd3, Wm, bm, Ws, bs, Wp, bp, mu)
    return pl.pallas_call(
        _heads_body,
        grid=grid,
        in_specs=[row_spec(LAT)] + [full(c) for c in consts],
        out_specs=(row_spec(ADJ_DIM), row_spec(NCLUST), row_spec(IN_DIM),
                   row_spec(IN_DIM), row_spec(IN_DIM)),
        out_shape=out_shapes,
    )(z, *consts)


def _adj_out(dec_h):
    grid = (N // ROWS_B,)
    return pl.pallas_call(
        _adj_body,
        grid=grid,
        in_specs=[pl.BlockSpec((ROWS_B, ADJ_DIM), lambda i: (i, 0)),
                  pl.BlockSpec((N, ADJ_DIM), lambda i: (0, 0))],
        out_specs=pl.BlockSpec((ROWS_B, N), lambda i: (i, 0)),
        out_shape=jax.ShapeDtypeStruct((N, N), jnp.float32),
    )(dec_h, dec_h)


def kernel(X_input, edge_index, W1, b1, W2, b2, Wa, ba, Wd1, bd1, Wd2, bd2,
           Wd3, bd3, Wm, bm, Ws, bs, Wp, bp, mu):
    src, dst = edge_index[0], edge_index[1]
    padlen = EP - E
    padidx = N + (jnp.arange(padlen, dtype=jnp.int32) % (NP - N))
    srcp = jnp.concatenate(
        [src.astype(jnp.int32), padidx]).reshape(NW * CHUNKS_W, EB)
    dstp = jnp.concatenate(
        [dst.astype(jnp.int32), padidx]).reshape(NW * CHUNKS_W, EB)

    zrows16 = jnp.zeros((ZROWS, 16), jnp.float32)

    dparts = _sc_degree(dstp, zrows16)
    deg = dparts[0, :N, 0] + dparts[1, :N, 0]
    norm = jnp.where(deg > 0.0, deg, 1.0) ** -0.5

    W1h = W1.reshape(K + 1, IN_DIM, HID)
    W2h = W2.reshape(K + 1, HID, LAT)
    KK = 2 * K
    Cs = {}
    for i in range(K + 1):
        for j in range(K + 1):
            kk = i + j
            prod = W1h[j] @ W2h[i]
            Cs[kk] = prod if kk not in Cs else Cs[kk] + prod
    colpad = jnp.zeros((IN_DIM, 16 - LAT), jnp.float32)
    Ccat = jnp.concatenate(
        sum(([Cs[kk], colpad] for kk in range(KK + 1)), []), axis=1)
    T = _mm(X_input, Ccat)
    Tp = jnp.concatenate(
        [T, jnp.zeros((NP - N, (KK + 1) * 16), jnp.float32)], axis=0)
    cvec = [jnp.pad(b1 @ W2h[i], (0, 16 - LAT)) for i in range(K + 1)]

    def injectp(kk):
        t = Tp[:, kk * 16:(kk + 1) * 16]
        return t + cvec[kk][None, :] if kk <= K else t

    normp = jnp.concatenate([norm, jnp.zeros((NP - N,), jnp.float32)])[:, None]
    norm2p = normp * normp
    c = normp * injectp(KK)
    for kk in range(KK - 1, 0, -1):
        parts = _sc_segsum(c, srcp, dstp, zrows16)
        c = norm2p * (parts[0] + parts[1]) + normp * injectp(kk)
    parts = _sc_segsum(c, srcp, dstp, zrows16)
    zfull = normp * (parts[0] + parts[1]) + injectp(0)
    z = zfull[:N, :LAT] + b2

    dec_h, q, _mean, _disp, _pi = _decoder_heads(
        z, Wa, ba, Wd1, bd1, Wd2, bd2, Wd3, bd3, Wm, bm, Ws, bs, Wp, bp, mu)
    A_out = _adj_out(dec_h)
    return (A_out, z, q, _mean, _disp, _pi)
